# Initial kernel scaffold; baseline (speedup 1.0000x reference)
import functools

import jax
import jax.numpy as jnp
from jax import lax
from jax.experimental import pallas as pl
from jax.experimental.pallas import tpu as pltpu
from jax.experimental.pallas import tpu_sc as plsc

B, L, D = 4096, 200, 64
NC, NS = 2, 16
NW = NC * NS
RPW = B // NW
TOK_W = RPW * L
ROW_W = L * D
PN = 2 * L * D

G0, G1 = 104, 96


def _mo(v):
    return pl.multiple_of(v, 8)


def kernel(x, word_table, pos_table):
    xf = x.reshape(B * L)
    p_tab = jnp.concatenate(
        [jnp.zeros((L, D), jnp.float32), pos_table[1:L + 1]], axis=0
    ).reshape(PN)

    mesh = plsc.VectorSubcoreMesh(core_axis_name="c", subcore_axis_name="s")

    @functools.partial(
        pl.kernel,
        out_type=jax.ShapeDtypeStruct((B * L, D), jnp.float32),
        mesh=mesh,
        scratch_types=[
            pltpu.VMEM((TOK_W + 8,), jnp.int32),
            pltpu.VMEM((L, D), jnp.float32),
            pltpu.VMEM((PN,), jnp.float32),
            pltpu.SemaphoreType.DMA,
        ],
    )
    def run(xf_hbm, wt_hbm, p_hbm, out_hbm, idx_v, buf_v, p_v, sem):
        wid = lax.axis_index("s") * NC + lax.axis_index("c")
        tok0 = _mo(wid * TOK_W)
        pltpu.sync_copy(p_hbm, p_v)
        pltpu.sync_copy(xf_hbm.at[pl.ds(tok0, TOK_W)], idx_v.at[pl.ds(0, TOK_W)])

        def row_body(i, carry):
            rbase = _mo(i * L)

            def cnt_body(k, acc):
                vals = idx_v[pl.ds(_mo(rbase + k * 16), 16)]
                valid = (k * 16 + lax.iota(jnp.int32, 16)) < L
                return acc + jnp.where(
                    jnp.logical_and(vals != jnp.int32(0), valid),
                    jnp.int32(1), jnp.int32(0))
            acc = lax.fori_loop(0, (L + 15) // 16, cnt_body,
                                jnp.zeros((16,), jnp.int32))
            ls = jnp.sum(acc)

            c1 = pltpu.async_copy(
                wt_hbm.at[idx_v.at[pl.ds(rbase, G0)]],
                buf_v.at[pl.ds(0, G0)], sem)
            c2 = pltpu.async_copy(
                wt_hbm.at[idx_v.at[pl.ds(_mo(rbase + G0), G1)]],
                buf_v.at[pl.ds(G0, G1)], sem)
            c1.wait()
            c2.wait()

            pbase = pl.multiple_of(ls * D, 16)

            def add_body(m, c):
                for r in range(D // 16):
                    sl = pl.ds(r * 16, 16)
                    buf_v[m, sl] = buf_v[m, sl] + p_v[
                        pl.ds(pbase + m * D + r * 16, 16)]
                return c
            lax.fori_loop(0, L, add_body, 0)

            pltpu.sync_copy(
                buf_v, out_hbm.at[pl.ds(_mo((wid * RPW + i) * L), L)])
            return carry

        lax.fori_loop(0, RPW, row_body, 0)

    out = run(xf, word_table, p_tab)
    return out.reshape(B, L, D)


# speedup vs baseline: 1.9167x; 1.9167x over previous
import functools

import jax
import jax.numpy as jnp
from jax import lax
from jax.experimental import pallas as pl
from jax.experimental.pallas import tpu as pltpu
from jax.experimental.pallas import tpu_sc as plsc

B, L, D = 4096, 200, 64
NC, NS = 2, 16
NW = NC * NS
RPW = B // NW
TOK_W = RPW * L
ROW_W = L * D
PN = 2 * L * D

G0, G1 = 104, 96


def _mo(v):
    return pl.multiple_of(v, 8)


def kernel(x, word_table, pos_table):
    xf = x.reshape(B * L)
    p_tab = jnp.concatenate(
        [jnp.zeros((L, D), jnp.float32), pos_table[1:L + 1]], axis=0
    ).reshape(PN)

    mesh = plsc.VectorSubcoreMesh(core_axis_name="c", subcore_axis_name="s")

    @functools.partial(
        pl.kernel,
        out_type=jax.ShapeDtypeStruct((B * L, D), jnp.float32),
        mesh=mesh,
        compiler_params=pltpu.CompilerParams(
            needs_layout_passes=False, use_tc_tiling_on_sc=False),
        scratch_types=[
            pltpu.VMEM((TOK_W + 8,), jnp.int32),
            pltpu.VMEM((L, D), jnp.float32),
            pltpu.VMEM((PN,), jnp.float32),
            pltpu.SemaphoreType.DMA,
        ],
    )
    def run(xf_hbm, wt_hbm, p_hbm, out_hbm, idx_v, buf_v, p_v, sem):
        wid = lax.axis_index("s") * NC + lax.axis_index("c")
        tok0 = _mo(wid * TOK_W)
        pltpu.sync_copy(p_hbm, p_v)
        pltpu.sync_copy(xf_hbm.at[pl.ds(tok0, TOK_W)], idx_v.at[pl.ds(0, TOK_W)])

        def row_body(i, carry):
            rbase = _mo(i * L)

            def cnt_body(k, acc):
                vals = idx_v[pl.ds(_mo(rbase + k * 16), 16)]
                valid = (k * 16 + lax.iota(jnp.int32, 16)) < L
                m = jnp.logical_and(vals != jnp.int32(0), valid)
                return acc + plsc.all_reduce_population_count(m)
            ls = lax.fori_loop(0, (L + 15) // 16, cnt_body,
                               jnp.zeros((16,), jnp.int32))

            c1 = pltpu.async_copy(
                wt_hbm.at[idx_v.at[pl.ds(rbase, G0)]],
                buf_v.at[pl.ds(0, G0)], sem)
            c2 = pltpu.async_copy(
                wt_hbm.at[idx_v.at[pl.ds(_mo(rbase + G0), G1)]],
                buf_v.at[pl.ds(G0, G1)], sem)
            c1.wait()
            c2.wait()

            pvec = ls * jnp.int32(D) + lax.iota(jnp.int32, 16)

            def add_body(m, c):
                for r in range(D // 16):
                    sl = pl.ds(r * 16, 16)
                    prow = plsc.load_gather(
                        p_v, [pvec + jnp.int32(m * D + r * 16)])
                    buf_v[m, sl] = buf_v[m, sl] + prow
                return c
            lax.fori_loop(0, L, add_body, 0)

            pltpu.sync_copy(
                buf_v, out_hbm.at[pl.ds(_mo((wid * RPW + i) * L), L)])
            return carry

        lax.fori_loop(0, RPW, row_body, 0)

    out = run(xf, word_table, p_tab)
    return out.reshape(B, L, D)


# speedup vs baseline: 2.8148x; 1.4686x over previous
import functools

import jax
import jax.numpy as jnp
from jax import lax
from jax.experimental import pallas as pl
from jax.experimental.pallas import tpu as pltpu
from jax.experimental.pallas import tpu_sc as plsc

B, L, D = 4096, 200, 64
NC, NS = 2, 16
NW = NC * NS
RPW = B // NW
TOK_W = RPW * L
PN = 2 * L * D

G0, G1 = 104, 96


def _mo(v):
    return pl.multiple_of(v, 8)


def kernel(x, word_table, pos_table):
    xf = x.reshape(B * L)
    p_tab = jnp.concatenate(
        [jnp.zeros((L, D), jnp.float32), pos_table[1:L + 1]], axis=0
    ).reshape(PN)

    mesh = plsc.VectorSubcoreMesh(core_axis_name="c", subcore_axis_name="s")

    @functools.partial(
        pl.kernel,
        out_type=jax.ShapeDtypeStruct((B * L, D), jnp.float32),
        mesh=mesh,
        compiler_params=pltpu.CompilerParams(
            needs_layout_passes=False, use_tc_tiling_on_sc=False),
        scratch_types=[
            pltpu.VMEM((TOK_W + 8,), jnp.int32),
            pltpu.VMEM((2 * L, D), jnp.float32),
            pltpu.VMEM((2 * L, D), jnp.float32),
            pltpu.VMEM((PN,), jnp.float32),
            pltpu.SemaphoreType.DMA,
            pltpu.SemaphoreType.DMA,
            pltpu.SemaphoreType.DMA,
            pltpu.SemaphoreType.DMA,
        ],
    )
    def run(xf_hbm, wt_hbm, p_hbm, out_hbm, idx_v, gbuf, sbuf, p_v,
            gsem0, gsem1, ssem0, ssem1):
        wid = lax.axis_index("s") * NC + lax.axis_index("c")
        tok0 = _mo(wid * TOK_W)
        pltpu.sync_copy(p_hbm, p_v)
        pltpu.sync_copy(xf_hbm.at[pl.ds(tok0, TOK_W)], idx_v.at[pl.ds(0, TOK_W)])

        gsems = (gsem0, gsem1)
        ssems = (ssem0, ssem1)

        def gather_descs(i, s):
            rbase = _mo(i * L)
            return (
                pltpu.make_async_copy(
                    wt_hbm.at[idx_v.at[pl.ds(rbase, G0)]],
                    gbuf.at[pl.ds(s * L, G0)], gsems[s]),
                pltpu.make_async_copy(
                    wt_hbm.at[idx_v.at[pl.ds(_mo(rbase + G0), G1)]],
                    gbuf.at[pl.ds(s * L + G0, G1)], gsems[s]),
            )

        def store_desc(i, s):
            return pltpu.make_async_copy(
                sbuf.at[pl.ds(s * L, L)],
                out_hbm.at[pl.ds(_mo((wid * RPW + i) * L), L)], ssems[s])

        def add_row(i, s):
            rbase = _mo(i * L)

            def cnt_body(k, acc):
                vals = idx_v[pl.ds(_mo(rbase + k * 16), 16)]
                valid = (k * 16 + lax.iota(jnp.int32, 16)) < L
                mask = jnp.logical_and(vals != jnp.int32(0), valid)
                return acc + plsc.all_reduce_population_count(mask)
            ls = lax.fori_loop(0, (L + 15) // 16, cnt_body,
                               jnp.zeros((16,), jnp.int32))
            pvec = ls * jnp.int32(D) + lax.iota(jnp.int32, 16)

            @plsc.parallel_loop(0, L, unroll=4)
            def _(m):
                for r in range(D // 16):
                    sl = pl.ds(r * 16, 16)
                    prow = plsc.load_gather(
                        p_v, [pvec + jnp.int32(m * D + r * 16)])
                    sbuf[s * L + m, sl] = gbuf[s * L + m, sl] + prow

        for s in range(2):
            for d in gather_descs(s, s):
                d.start()

        def pair_body(g, carry):
            for s in range(2):
                i = 2 * g + s

                @pl.when(g > 0)
                def _():
                    store_desc(0, s).wait()

                for d in gather_descs(i, s):
                    d.wait()

                add_row(i, s)
                store_desc(i, s).start()

                @pl.when(g < RPW // 2 - 1)
                def _():
                    for d in gather_descs(i + 2, s):
                        d.start()
            return carry

        lax.fori_loop(0, RPW // 2, pair_body, 0)
        for s in range(2):
            store_desc(0, s).wait()

    out = run(xf, word_table, p_tab)
    return out.reshape(B, L, D)
